# Initial kernel scaffold; baseline (speedup 1.0000x reference)
import jax
import jax.numpy as jnp
from jax.experimental import pallas as pl


def kernel(emb0, emb1, emb2, weight):
    raise NotImplementedError("write your pallas kernel here")


# speedup vs baseline: 2.3600x; 2.3600x over previous
import jax
import jax.numpy as jnp
from jax.experimental import pallas as pl

_N = 100000
_D = 256
_BLOCK = 1000


def _fuse_block(w_ref, e0_ref, e1_ref, e2_ref, out_ref):
    w = w_ref[:]
    e = jnp.exp(w - jnp.max(w))
    wn = e / jnp.sum(e)
    for i, ref in enumerate((e0_ref, e1_ref, e2_ref)):
        x = ref[:]
        n = jnp.sqrt(jnp.sum(x * x, axis=1, keepdims=True))
        out_ref[:, i * _D:(i + 1) * _D] = x / jnp.maximum(n, 1e-12) * wn[i]


def kernel(emb0, emb1, emb2, weight):
    n, d = emb0.shape
    grid = (n // _BLOCK,)
    emb_spec = pl.BlockSpec((_BLOCK, d), lambda i: (i, 0))
    return pl.pallas_call(
        _fuse_block,
        grid=grid,
        in_specs=[
            pl.BlockSpec((3, 1), lambda i: (0, 0)),
            emb_spec, emb_spec, emb_spec,
        ],
        out_specs=pl.BlockSpec((_BLOCK, 3 * d), lambda i: (i, 0)),
        out_shape=jax.ShapeDtypeStruct((n, 3 * d), emb0.dtype),
    )(weight, emb0, emb1, emb2)


# speedup vs baseline: 2.4906x; 1.0553x over previous
import jax
import jax.numpy as jnp
from jax.experimental import pallas as pl

_N = 100000
_D = 256
_BLOCK = 4000


def _fuse_block(w_ref, e0_ref, e1_ref, e2_ref, out_ref):
    w = w_ref[:]
    e = jnp.exp(w - jnp.max(w))
    wn = e / jnp.sum(e)
    for i, ref in enumerate((e0_ref, e1_ref, e2_ref)):
        x = ref[:]
        n = jnp.sqrt(jnp.sum(x * x, axis=1, keepdims=True))
        out_ref[:, i * _D:(i + 1) * _D] = x / jnp.maximum(n, 1e-12) * wn[i]


def kernel(emb0, emb1, emb2, weight):
    n, d = emb0.shape
    grid = (n // _BLOCK,)
    emb_spec = pl.BlockSpec((_BLOCK, d), lambda i: (i, 0))
    return pl.pallas_call(
        _fuse_block,
        grid=grid,
        in_specs=[
            pl.BlockSpec((3, 1), lambda i: (0, 0)),
            emb_spec, emb_spec, emb_spec,
        ],
        out_specs=pl.BlockSpec((_BLOCK, 3 * d), lambda i: (i, 0)),
        out_shape=jax.ShapeDtypeStruct((n, 3 * d), emb0.dtype),
    )(weight, emb0, emb1, emb2)
